# baseline (speedup 1.0000x reference)
import functools

import jax
import jax.numpy as jnp
from jax import lax
from jax.experimental import pallas as pl
from jax.experimental.pallas import tpu as pltpu
from jax.experimental.pallas import tpu_sc as plsc

N = 10000
E = 320000
D = 128
H = 256
C = 10
G = 64

NC = 2
NS = 16
DH = 128
FB = 128
SLB = 632
SLL = N - (NS - 1) * SLB

_BN_SCALE = float(1.0 / (1.0 + 1e-5) ** 0.5)


@functools.lru_cache(maxsize=None)
def _make_agg(mode):
    eoff = E // NC if mode == "edge" else 0
    et = (E // NC if mode == "edge" else E) // NS
    fb = 64
    nbf = et // fb
    tail = et - nbf * fb
    K = 4
    ng = nbf // K
    rem = nbf - ng * K
    mesh = plsc.VectorSubcoreMesh(core_axis_name="c", subcore_axis_name="s",
                                  num_cores=NC, num_subcores=NS)

    @functools.partial(
        pl.kernel,
        out_type=[jax.ShapeDtypeStruct((N, DH), jnp.float32),
                  jax.ShapeDtypeStruct((N, DH), jnp.float32)],
        mesh=mesh,
        scratch_types=[
            pltpu.VMEM((2, K, fb), jnp.int32),
            pltpu.VMEM((2, K, fb), jnp.int32),
            pltpu.VMEM((K, fb, DH), jnp.float32),
            pltpu.VMEM((tail,), jnp.int32),
            pltpu.VMEM((tail,), jnp.int32),
            pltpu.VMEM((tail, DH), jnp.float32),
            pltpu.VMEM_SHARED((N, DH), jnp.float32),
            pltpu.SemaphoreType.DMA,
            pltpu.SemaphoreType.DMA,
            pltpu.SemaphoreType.DMA,
        ],
    )
    def agg_kernel(t0_hbm, t1_hbm, i0_hbm, i1_hbm, src_hbm, dst_hbm,
                   out0_hbm, out1_hbm,
                   srcw, dstw, rows, src_t, dst_t, rows_t, acc,
                   sem_d, sem_g, sem_s):
        c = lax.axis_index("c")
        s = lax.axis_index("s")
        e0 = c * eoff + s * et
        r0 = s * SLB

        def run(t_hbm, i_hbm, out_hbm):
            @pl.when(s < NS - 1)
            def _():
                pltpu.sync_copy(i_hbm.at[pl.ds(r0, SLB)],
                                acc.at[pl.ds(r0, SLB)])

            @pl.when(s == NS - 1)
            def _():
                pltpu.sync_copy(i_hbm.at[pl.ds(r0, SLL)],
                                acc.at[pl.ds(r0, SLL)])

            plsc.subcore_barrier()

            def fire_idx(b0, p):
                cps = []
                for j in range(K):
                    off = e0 + (b0 + j) * fb
                    cps.append(pltpu.async_copy(
                        src_hbm.at[pl.ds(off, fb)], srcw.at[p, j], sem_d))
                    cps.append(pltpu.async_copy(
                        dst_hbm.at[pl.ds(off, fb)], dstw.at[p, j], sem_d))
                return cps

            def do_group(p):
                gcs = [pltpu.async_copy(
                    t_hbm.at[srcw.at[p, 0]], rows.at[0], sem_g)]
                scs = []
                for j in range(K):
                    gcs[j].wait()
                    if j + 1 < K:
                        gcs.append(pltpu.async_copy(
                            t_hbm.at[srcw.at[p, j + 1]], rows.at[j + 1],
                            sem_g))
                    scs.append(pltpu.async_copy(
                        rows.at[j], acc.at[dstw.at[p, j]], sem_s, add=True))
                for cp in scs:
                    cp.wait()

            def pair(i, carry):
                b0 = i * 2 * K
                c0 = fire_idx(b0, 0)
                c1 = fire_idx(b0 + K, 1)
                for cp in c0:
                    cp.wait()
                do_group(0)
                for cp in c1:
                    cp.wait()
                do_group(1)
                return carry

            npair = ng // 2
            lax.fori_loop(0, npair, pair, jnp.int32(0))

            if ng % 2:
                b0 = npair * 2 * K
                for cp in fire_idx(b0, 0):
                    cp.wait()
                do_group(0)

            for r in range(rem):
                off = e0 + (ng * K + r) * fb
                pltpu.sync_copy(src_hbm.at[pl.ds(off, fb)], srcw.at[0, 0])
                pltpu.sync_copy(dst_hbm.at[pl.ds(off, fb)], dstw.at[0, 0])
                pltpu.async_copy(t_hbm.at[srcw.at[0, 0]], rows.at[0],
                                 sem_g).wait()
                pltpu.sync_copy(rows.at[0], acc.at[dstw.at[0, 0]], add=True)

            off = e0 + nbf * fb
            pltpu.sync_copy(src_hbm.at[pl.ds(off, tail)], src_t)
            pltpu.sync_copy(dst_hbm.at[pl.ds(off, tail)], dst_t)
            pltpu.async_copy(t_hbm.at[src_t], rows_t, sem_g).wait()
            pltpu.sync_copy(rows_t, acc.at[dst_t], add=True)

            plsc.subcore_barrier()

            @pl.when(s < NS - 1)
            def _():
                pltpu.sync_copy(acc.at[pl.ds(r0, SLB)],
                                out_hbm.at[pl.ds(r0, SLB)])

            @pl.when(s == NS - 1)
            def _():
                pltpu.sync_copy(acc.at[pl.ds(r0, SLL)],
                                out_hbm.at[pl.ds(r0, SLL)])

        @pl.when(c == 0)
        def _():
            run(t0_hbm, i0_hbm, out0_hbm)

        @pl.when(c == 1)
        def _():
            run(t1_hbm, i1_hbm, out1_hbm)

    return agg_kernel


def _mlp_math(a0_ref, a1_ref, w1a_ref, w1b_ref, b1_ref, w2_ref, b2_ref,
              g_ref, bt_ref):
    z = (jnp.dot(a0_ref[...], w1a_ref[...],
                 preferred_element_type=jnp.float32)
         + jnp.dot(a1_ref[...], w1b_ref[...],
                   preferred_element_type=jnp.float32)
         + b1_ref[...])
    z = jnp.maximum(z, 0.0)
    z = jnp.maximum(
        jnp.dot(z, w2_ref[...], preferred_element_type=jnp.float32)
        + b2_ref[...], 0.0)
    return g_ref[...] * (z * _BN_SCALE) + bt_ref[...]


def _mlp_split_body(a0_ref, a1_ref, w1a_ref, w1b_ref, b1_ref, w2_ref, b2_ref,
                    g_ref, bt_ref, o0_ref, o1_ref):
    r = _mlp_math(a0_ref, a1_ref, w1a_ref, w1b_ref, b1_ref, w2_ref, b2_ref,
                  g_ref, bt_ref)
    o0_ref[...] = r[:, :DH]
    o1_ref[...] = r[:, DH:]


def _mlp_full_body(a0_ref, a1_ref, w1a_ref, w1b_ref, b1_ref, w2_ref, b2_ref,
                   g_ref, bt_ref, o_ref):
    o_ref[...] = _mlp_math(a0_ref, a1_ref, w1a_ref, w1b_ref, b1_ref, w2_ref,
                           b2_ref, g_ref, bt_ref)


@functools.lru_cache(maxsize=None)
def _make_mlp(split_out):
    rb = 1000
    grid = (N // rb,)
    in_specs = [
        pl.BlockSpec((rb, DH), lambda i: (i, 0)),
        pl.BlockSpec((rb, DH), lambda i: (i, 0)),
        pl.BlockSpec((DH, H), lambda i: (0, 0)),
        pl.BlockSpec((DH, H), lambda i: (0, 0)),
        pl.BlockSpec((1, H), lambda i: (0, 0)),
        pl.BlockSpec((H, H), lambda i: (0, 0)),
        pl.BlockSpec((1, H), lambda i: (0, 0)),
        pl.BlockSpec((1, H), lambda i: (0, 0)),
        pl.BlockSpec((1, H), lambda i: (0, 0)),
    ]
    if split_out:
        return pl.pallas_call(
            _mlp_split_body,
            grid=grid,
            in_specs=in_specs,
            out_specs=[pl.BlockSpec((rb, DH), lambda i: (i, 0)),
                       pl.BlockSpec((rb, DH), lambda i: (i, 0))],
            out_shape=[jax.ShapeDtypeStruct((N, DH), jnp.float32),
                       jax.ShapeDtypeStruct((N, DH), jnp.float32)],
        )
    return pl.pallas_call(
        _mlp_full_body,
        grid=grid,
        in_specs=in_specs,
        out_specs=pl.BlockSpec((rb, H), lambda i: (i, 0)),
        out_shape=jax.ShapeDtypeStruct((N, H), jnp.float32),
    )


def _head_body(emb_ref, b3_ref, w1_ref, b1_ref, w2_ref, b2_ref, o_ref,
               acc, cnt):
    i = pl.program_id(0)
    rb = emb_ref.shape[0]

    @pl.when(i == 0)
    def _():
        acc[...] = jnp.zeros_like(acc)
        cnt[...] = jnp.zeros_like(cnt)

    bvec = b3_ref[0]
    gids = lax.broadcasted_iota(jnp.int32, (G, rb), 0)
    onehot = (bvec == gids).astype(jnp.float32)
    acc[...] += jnp.dot(onehot, emb_ref[...],
                        preferred_element_type=jnp.float32)
    cnt[...] += jnp.broadcast_to(jnp.sum(onehot, axis=1, keepdims=True),
                                 cnt.shape)

    @pl.when(i == pl.num_programs(0) - 1)
    def _():
        pooled = acc[...] / jnp.maximum(cnt[...], 1.0)
        z = jnp.dot(pooled, w1_ref[...],
                    preferred_element_type=jnp.float32) + b1_ref[...]
        z = jnp.dot(z, w2_ref[...],
                    preferred_element_type=jnp.float32) + b2_ref[...]
        z = jnp.clip(z, -10.0, 10.0)
        m = jnp.max(z, axis=-1, keepdims=True)
        lse = m + jnp.log(jnp.sum(jnp.exp(z - m), axis=-1, keepdims=True))
        o_ref[...] = z - lse


def _make_head():
    rb = 1000
    grid = (N // rb,)
    return pl.pallas_call(
        _head_body,
        grid=grid,
        in_specs=[
            pl.BlockSpec((rb, H), lambda i: (i, 0)),
            pl.BlockSpec((1, 1, rb), lambda i: (i, 0, 0)),
            pl.BlockSpec((H, H), lambda i: (0, 0)),
            pl.BlockSpec((1, H), lambda i: (0, 0)),
            pl.BlockSpec((H, C), lambda i: (0, 0)),
            pl.BlockSpec((1, C), lambda i: (0, 0)),
        ],
        out_specs=pl.BlockSpec((G, C), lambda i: (0, 0)),
        out_shape=jax.ShapeDtypeStruct((G, C), jnp.float32),
        scratch_shapes=[
            pltpu.VMEM((G, H), jnp.float32),
            pltpu.VMEM((G, H), jnp.float32),
        ],
    )


def kernel(x, edge_index, batch, params):
    src = edge_index[0]
    dst = edge_index[1]
    layers = params["layers"]

    zeros = jnp.zeros((N, DH), jnp.float32)
    p0, p1 = _make_agg("edge")(x, x, x, zeros, src, dst)
    w1 = layers[0]["W1"]
    h0, h1 = _make_mlp(True)(
        p0, p1, w1, w1, layers[0]["b1"][None], layers[0]["W2"],
        layers[0]["b2"][None], layers[0]["gamma"][None],
        layers[0]["beta"][None])

    for li in (1, 2):
        p = layers[li]
        a0, a1 = _make_agg("feat")(h0, h1, h0, h1, src, dst)
        w1a = p["W1"][:DH]
        w1b = p["W1"][DH:]
        if li < 2:
            h0, h1 = _make_mlp(True)(
                a0, a1, w1a, w1b, p["b1"][None], p["W2"], p["b2"][None],
                p["gamma"][None], p["beta"][None])
        else:
            embeds = _make_mlp(False)(
                a0, a1, w1a, w1b, p["b1"][None], p["W2"], p["b2"][None],
                p["gamma"][None], p["beta"][None])

    batch3 = batch.reshape(N // 1000, 1, 1000)
    out = _make_head()(
        embeds, batch3, params["lin1_W"], params["lin1_b"][None],
        params["lin2_W"], params["lin2_b"][None])
    return out, embeds


# speedup vs baseline: 1.4699x; 1.4699x over previous
import functools

import jax
import jax.numpy as jnp
from jax import lax
from jax.experimental import pallas as pl
from jax.experimental.pallas import tpu as pltpu
from jax.experimental.pallas import tpu_sc as plsc

N = 10000
E = 320000
D = 128
H = 256
C = 10
G = 64

NC = 2
NS = 16
DH = 128
FB = 128
SLB = 632
SLL = N - (NS - 1) * SLB

_BN_SCALE = float(1.0 / (1.0 + 1e-5) ** 0.5)


@functools.lru_cache(maxsize=None)
def _make_agg(mode):
    eoff = E // NC if mode == "edge" else 0
    et = (E // NC if mode == "edge" else E) // NS
    fb = 64
    nbf = et // fb
    tail = et - nbf * fb
    K = 4
    ng = nbf // K
    rem = nbf - ng * K
    mesh = plsc.VectorSubcoreMesh(core_axis_name="c", subcore_axis_name="s",
                                  num_cores=NC, num_subcores=NS)

    @functools.partial(
        pl.kernel,
        out_type=[jax.ShapeDtypeStruct((N, DH), jnp.float32),
                  jax.ShapeDtypeStruct((N, DH), jnp.float32)],
        mesh=mesh,
        scratch_types=[
            pltpu.VMEM((2, K, fb), jnp.int32),
            pltpu.VMEM((2, K, fb), jnp.int32),
            pltpu.VMEM((K, fb, DH), jnp.float32),
            pltpu.VMEM((tail,), jnp.int32),
            pltpu.VMEM((tail,), jnp.int32),
            pltpu.VMEM((tail, DH), jnp.float32),
            pltpu.VMEM_SHARED((N, DH), jnp.float32),
            pltpu.SemaphoreType.DMA,
            pltpu.SemaphoreType.DMA,
            pltpu.SemaphoreType.DMA,
        ],
    )
    def agg_kernel(t0_hbm, t1_hbm, i0_hbm, i1_hbm, src_hbm, dst_hbm,
                   out0_hbm, out1_hbm,
                   srcw, dstw, rows, src_t, dst_t, rows_t, acc,
                   sem_d, sem_g, sem_s):
        c = lax.axis_index("c")
        s = lax.axis_index("s")
        e0 = c * eoff + s * et
        r0 = s * SLB

        def run(t_hbm, i_hbm, out_hbm):
            @pl.when(s < NS - 1)
            def _():
                pltpu.sync_copy(i_hbm.at[pl.ds(r0, SLB)],
                                acc.at[pl.ds(r0, SLB)])

            @pl.when(s == NS - 1)
            def _():
                pltpu.sync_copy(i_hbm.at[pl.ds(r0, SLL)],
                                acc.at[pl.ds(r0, SLL)])

            plsc.subcore_barrier()

            def fire_idx(b0, p):
                cps = []
                for j in range(K):
                    off = e0 + (b0 + j) * fb
                    cps.append(pltpu.async_copy(
                        src_hbm.at[pl.ds(off, fb)], srcw.at[p, j], sem_d))
                    cps.append(pltpu.async_copy(
                        dst_hbm.at[pl.ds(off, fb)], dstw.at[p, j], sem_d))
                return cps

            def do_group(p):
                gcs = []
                for j in range(K):
                    gcs.append(pltpu.async_copy(
                        t_hbm.at[srcw.at[p, j]], rows.at[j], sem_g))
                scs = []
                for j in range(K):
                    gcs[j].wait()
                    scs.append(pltpu.async_copy(
                        rows.at[j], acc.at[dstw.at[p, j]], sem_s, add=True))
                for cp in scs:
                    cp.wait()

            def pair(i, carry):
                b0 = i * 2 * K
                c0 = fire_idx(b0, 0)
                c1 = fire_idx(b0 + K, 1)
                for cp in c0:
                    cp.wait()
                do_group(0)
                for cp in c1:
                    cp.wait()
                do_group(1)
                return carry

            npair = ng // 2
            lax.fori_loop(0, npair, pair, jnp.int32(0))

            if ng % 2:
                b0 = npair * 2 * K
                for cp in fire_idx(b0, 0):
                    cp.wait()
                do_group(0)

            for r in range(rem):
                off = e0 + (ng * K + r) * fb
                pltpu.sync_copy(src_hbm.at[pl.ds(off, fb)], srcw.at[0, 0])
                pltpu.sync_copy(dst_hbm.at[pl.ds(off, fb)], dstw.at[0, 0])
                pltpu.async_copy(t_hbm.at[srcw.at[0, 0]], rows.at[0],
                                 sem_g).wait()
                pltpu.sync_copy(rows.at[0], acc.at[dstw.at[0, 0]], add=True)

            off = e0 + nbf * fb
            pltpu.sync_copy(src_hbm.at[pl.ds(off, tail)], src_t)
            pltpu.sync_copy(dst_hbm.at[pl.ds(off, tail)], dst_t)
            pltpu.async_copy(t_hbm.at[src_t], rows_t, sem_g).wait()
            pltpu.sync_copy(rows_t, acc.at[dst_t], add=True)

            plsc.subcore_barrier()

            @pl.when(s < NS - 1)
            def _():
                pltpu.sync_copy(acc.at[pl.ds(r0, SLB)],
                                out_hbm.at[pl.ds(r0, SLB)])

            @pl.when(s == NS - 1)
            def _():
                pltpu.sync_copy(acc.at[pl.ds(r0, SLL)],
                                out_hbm.at[pl.ds(r0, SLL)])

        @pl.when(c == 0)
        def _():
            run(t0_hbm, i0_hbm, out0_hbm)

        @pl.when(c == 1)
        def _():
            run(t1_hbm, i1_hbm, out1_hbm)

    return agg_kernel


def _mlp_math(a0_ref, a1_ref, w1a_ref, w1b_ref, b1_ref, w2_ref, b2_ref,
              g_ref, bt_ref):
    z = (jnp.dot(a0_ref[...], w1a_ref[...],
                 preferred_element_type=jnp.float32)
         + jnp.dot(a1_ref[...], w1b_ref[...],
                   preferred_element_type=jnp.float32)
         + b1_ref[...])
    z = jnp.maximum(z, 0.0)
    z = jnp.maximum(
        jnp.dot(z, w2_ref[...], preferred_element_type=jnp.float32)
        + b2_ref[...], 0.0)
    return g_ref[...] * (z * _BN_SCALE) + bt_ref[...]


def _mlp_split_body(a0_ref, a1_ref, w1a_ref, w1b_ref, b1_ref, w2_ref, b2_ref,
                    g_ref, bt_ref, o0_ref, o1_ref):
    r = _mlp_math(a0_ref, a1_ref, w1a_ref, w1b_ref, b1_ref, w2_ref, b2_ref,
                  g_ref, bt_ref)
    o0_ref[...] = r[:, :DH]
    o1_ref[...] = r[:, DH:]


def _mlp_full_body(a0_ref, a1_ref, w1a_ref, w1b_ref, b1_ref, w2_ref, b2_ref,
                   g_ref, bt_ref, o_ref):
    o_ref[...] = _mlp_math(a0_ref, a1_ref, w1a_ref, w1b_ref, b1_ref, w2_ref,
                           b2_ref, g_ref, bt_ref)


@functools.lru_cache(maxsize=None)
def _make_mlp(split_out):
    rb = 1000
    grid = (N // rb,)
    in_specs = [
        pl.BlockSpec((rb, DH), lambda i: (i, 0)),
        pl.BlockSpec((rb, DH), lambda i: (i, 0)),
        pl.BlockSpec((DH, H), lambda i: (0, 0)),
        pl.BlockSpec((DH, H), lambda i: (0, 0)),
        pl.BlockSpec((1, H), lambda i: (0, 0)),
        pl.BlockSpec((H, H), lambda i: (0, 0)),
        pl.BlockSpec((1, H), lambda i: (0, 0)),
        pl.BlockSpec((1, H), lambda i: (0, 0)),
        pl.BlockSpec((1, H), lambda i: (0, 0)),
    ]
    if split_out:
        return pl.pallas_call(
            _mlp_split_body,
            grid=grid,
            in_specs=in_specs,
            out_specs=[pl.BlockSpec((rb, DH), lambda i: (i, 0)),
                       pl.BlockSpec((rb, DH), lambda i: (i, 0))],
            out_shape=[jax.ShapeDtypeStruct((N, DH), jnp.float32),
                       jax.ShapeDtypeStruct((N, DH), jnp.float32)],
        )
    return pl.pallas_call(
        _mlp_full_body,
        grid=grid,
        in_specs=in_specs,
        out_specs=pl.BlockSpec((rb, H), lambda i: (i, 0)),
        out_shape=jax.ShapeDtypeStruct((N, H), jnp.float32),
    )


def _head_body(emb_ref, b3_ref, w1_ref, b1_ref, w2_ref, b2_ref, o_ref,
               acc, cnt):
    i = pl.program_id(0)
    rb = emb_ref.shape[0]

    @pl.when(i == 0)
    def _():
        acc[...] = jnp.zeros_like(acc)
        cnt[...] = jnp.zeros_like(cnt)

    bvec = b3_ref[0]
    gids = lax.broadcasted_iota(jnp.int32, (G, rb), 0)
    onehot = (bvec == gids).astype(jnp.float32)
    acc[...] += jnp.dot(onehot, emb_ref[...],
                        preferred_element_type=jnp.float32)
    cnt[...] += jnp.broadcast_to(jnp.sum(onehot, axis=1, keepdims=True),
                                 cnt.shape)

    @pl.when(i == pl.num_programs(0) - 1)
    def _():
        pooled = acc[...] / jnp.maximum(cnt[...], 1.0)
        z = jnp.dot(pooled, w1_ref[...],
                    preferred_element_type=jnp.float32) + b1_ref[...]
        z = jnp.dot(z, w2_ref[...],
                    preferred_element_type=jnp.float32) + b2_ref[...]
        z = jnp.clip(z, -10.0, 10.0)
        m = jnp.max(z, axis=-1, keepdims=True)
        lse = m + jnp.log(jnp.sum(jnp.exp(z - m), axis=-1, keepdims=True))
        o_ref[...] = z - lse


def _make_head():
    rb = 1000
    grid = (N // rb,)
    return pl.pallas_call(
        _head_body,
        grid=grid,
        in_specs=[
            pl.BlockSpec((rb, H), lambda i: (i, 0)),
            pl.BlockSpec((1, 1, rb), lambda i: (i, 0, 0)),
            pl.BlockSpec((H, H), lambda i: (0, 0)),
            pl.BlockSpec((1, H), lambda i: (0, 0)),
            pl.BlockSpec((H, C), lambda i: (0, 0)),
            pl.BlockSpec((1, C), lambda i: (0, 0)),
        ],
        out_specs=pl.BlockSpec((G, C), lambda i: (0, 0)),
        out_shape=jax.ShapeDtypeStruct((G, C), jnp.float32),
        scratch_shapes=[
            pltpu.VMEM((G, H), jnp.float32),
            pltpu.VMEM((G, H), jnp.float32),
        ],
    )


def kernel(x, edge_index, batch, params):
    src = edge_index[0]
    dst = edge_index[1]
    layers = params["layers"]

    zeros = jnp.zeros((N, DH), jnp.float32)
    p0, p1 = _make_agg("edge")(x, x, x, zeros, src, dst)
    w1 = layers[0]["W1"]
    h0, h1 = _make_mlp(True)(
        p0, p1, w1, w1, layers[0]["b1"][None], layers[0]["W2"],
        layers[0]["b2"][None], layers[0]["gamma"][None],
        layers[0]["beta"][None])

    for li in (1, 2):
        p = layers[li]
        a0, a1 = _make_agg("feat")(h0, h1, h0, h1, src, dst)
        w1a = p["W1"][:DH]
        w1b = p["W1"][DH:]
        if li < 2:
            h0, h1 = _make_mlp(True)(
                a0, a1, w1a, w1b, p["b1"][None], p["W2"], p["b2"][None],
                p["gamma"][None], p["beta"][None])
        else:
            embeds = _make_mlp(False)(
                a0, a1, w1a, w1b, p["b1"][None], p["W2"], p["b2"][None],
                p["gamma"][None], p["beta"][None])

    batch3 = batch.reshape(N // 1000, 1, 1000)
    out = _make_head()(
        embeds, batch3, params["lin1_W"], params["lin1_b"][None],
        params["lin2_W"], params["lin2_b"][None])
    return out, embeds
